# Initial kernel scaffold; baseline (speedup 1.0000x reference)
import jax
import jax.numpy as jnp
from jax.experimental import pallas as pl


def kernel(z, mask):
    raise NotImplementedError("write your pallas kernel here")


# speedup vs baseline: 1.0006x; 1.0006x over previous
import jax
import jax.numpy as jnp
from jax.experimental import pallas as pl
from jax.experimental.pallas import tpu as pltpu

_ROWS_PER_BLOCK = 512


def _mask_mul_body(z_ref, m_ref, o_ref):
    o_ref[...] = z_ref[...] * m_ref[...]


def kernel(z, mask):
    B, S, D = z.shape
    rows = B * S
    z2 = z.reshape(rows, D)
    m2 = mask.reshape(1, D)
    grid = (rows // _ROWS_PER_BLOCK,)
    out = pl.pallas_call(
        _mask_mul_body,
        grid=grid,
        in_specs=[
            pl.BlockSpec((_ROWS_PER_BLOCK, D), lambda i: (i, 0)),
            pl.BlockSpec((1, D), lambda i: (0, 0)),
        ],
        out_specs=pl.BlockSpec((_ROWS_PER_BLOCK, D), lambda i: (i, 0)),
        out_shape=jax.ShapeDtypeStruct((rows, D), z.dtype),
    )(z2, m2)
    return out.reshape(B, S, D)
